# baseline (speedup 1.0000x reference)
import jax
import jax.numpy as jnp
from jax.experimental import pallas as pl
from jax.experimental.pallas import tpu as pltpu

S, H, I, E, K = 2048, 768, 3072, 8, 2
T = 128
NT = (S * K + E * (T - 1) + T - 1) // T
P = NT * T
TS = 256


def _prologue(hs_ref, g_ref, b_ref, rw_ref, rb_ref,
              xn_ref, pos_ref, wgt_ref, te_ref):
    x = hs_ref[...]
    mu = jnp.mean(x, axis=1, keepdims=True)
    xc = x - mu
    var = jnp.mean(xc * xc, axis=1, keepdims=True)
    xn = xc / jnp.sqrt(var + 1e-5) * g_ref[...] + b_ref[...]
    xn_ref[...] = xn.astype(jnp.bfloat16)

    logits = jax.lax.dot_general(
        rw_ref[...], xn, (((1,), (1,)), ((), ())),
        preferred_element_type=jnp.float32,
        precision=jax.lax.Precision.DEFAULT) + rb_ref[...]

    eio = jax.lax.broadcasted_iota(jnp.int32, (E, S), 0)
    m1 = jnp.max(logits, axis=0, keepdims=True)
    i1 = jnp.min(jnp.where(logits == m1, eio, E), axis=0, keepdims=True)
    masked = jnp.where(eio == i1, -1e30, logits)
    m2 = jnp.max(masked, axis=0, keepdims=True)
    i2 = jnp.min(jnp.where(masked == m2, eio, E), axis=0, keepdims=True)
    d = jnp.exp(m2 - m1)
    w_top1 = 1.0 / (1.0 + d)
    wgt_ref[...] = jnp.concatenate([w_top1, d * w_top1], axis=0)

    qio = jax.lax.broadcasted_iota(jnp.int32, (2 * E, S), 0)
    sel = jnp.where(qio < E,
                    jnp.broadcast_to(i1, (2 * E, S)),
                    jnp.broadcast_to(i2, (2 * E, S)))
    m = sel == qio % E

    sio = jax.lax.broadcasted_iota(jnp.int32, (S, S), 0)
    lio = jax.lax.broadcasted_iota(jnp.int32, (S, S), 1)
    tril = (sio <= lio).astype(jnp.bfloat16)
    incl = jax.lax.dot_general(m.astype(jnp.bfloat16), tril,
                               (((1,), (0,)), ((), ())),
                               preferred_element_type=jnp.float32)
    cnt_q = jnp.sum(m.astype(jnp.float32), axis=1, keepdims=True)
    cnt0 = cnt_q[0:E]
    cnts = cnt0 + cnt_q[E:]
    padded = jnp.floor((cnts + (T - 1)) / T) * T
    a8 = jax.lax.broadcasted_iota(jnp.int32, (E, E), 0)
    b8 = jax.lax.broadcasted_iota(jnp.int32, (E, E), 1)
    lstrict = (b8 < a8).astype(jnp.float32)
    start = jax.lax.dot_general(lstrict, padded, (((1,), (0,)), ((), ())),
                                preferred_element_type=jnp.float32)

    base_q = jnp.concatenate([start, start + cnt0], axis=0)
    field = jnp.where(m, base_q + incl - 1.0, 0.0)
    pos0 = jnp.sum(field[0:E], axis=0, keepdims=True)
    pos1 = jnp.sum(field[E:], axis=0, keepdims=True)
    pos_ref[...] = jnp.concatenate([pos0, pos1], axis=0).astype(jnp.int32)

    total = jnp.sum(padded, axis=0, keepdims=True) / T
    tio = (jax.lax.broadcasted_iota(jnp.int32, (1, 128), 1) * T).astype(jnp.float32)
    te = jnp.sum((start + padded <= tio).astype(jnp.int32), axis=0, keepdims=True)
    te = jnp.minimum(te, E - 1)
    lane = jax.lax.broadcasted_iota(jnp.int32, (1, 128), 1)
    te_ref[...] = jnp.where(lane == 127, total.astype(jnp.int32), te)


def _ffn(s_ref, xn_ref, pos_ref, w1_ref, b1_ref, w2_ref, b2_ref, y_ref,
         w1c_ref, w2c_ref):
    t = pl.program_id(0)
    used = s_ref[127]
    changed = jnp.logical_or(t == 0,
                             s_ref[t] != s_ref[jnp.maximum(t - 1, 0)])

    @pl.when(jnp.logical_and(changed, t < used))
    def _cast():
        w1c_ref[...] = w1_ref[0].astype(jnp.bfloat16)
        w2c_ref[...] = w2_ref[0].astype(jnp.bfloat16)

    @pl.when(t < used)
    def _compute():
        rows = jax.lax.broadcasted_iota(jnp.int32, (T, 1), 0) + t * T
        pos0 = pos_ref[0:1, :]
        pos1 = pos_ref[1:2, :]
        oh = ((pos0 == rows) | (pos1 == rows)).astype(jnp.bfloat16)
        xg = jax.lax.dot_general(oh, xn_ref[...], (((1,), (0,)), ((), ())),
                                 preferred_element_type=jnp.float32)
        h = jax.lax.dot_general(xg.astype(jnp.bfloat16), w1c_ref[...],
                                (((1,), (1,)), ((), ())),
                                preferred_element_type=jnp.float32)
        h = h + b1_ref[0]
        h = 0.5 * h * (1.0 + jax.lax.erf(h * 0.7071067811865476))
        o = jax.lax.dot_general(h.astype(jnp.bfloat16), w2c_ref[...],
                                (((1,), (1,)), ((), ())),
                                preferred_element_type=jnp.float32)
        y_ref[...] = (o + b2_ref[0]).astype(jnp.bfloat16)

    @pl.when(t >= used)
    def _zero():
        y_ref[...] = jnp.zeros((T, H), jnp.bfloat16)


def _combine(hs_ref, y_ref, post_ref, wgtt_ref, o_ref):
    p0 = post_ref[:, 0:1]
    p1 = post_ref[:, 1:2]
    w0 = wgtt_ref[:, 0:1]
    w1 = wgtt_ref[:, 1:2]
    cio = jax.lax.broadcasted_iota(jnp.int32, (TS, P), 1)
    comb = jnp.where(p0 == cio, w0, 0.0) + jnp.where(p1 == cio, w1, 0.0)
    acc = jax.lax.dot_general(comb.astype(jnp.bfloat16), y_ref[...],
                              (((1,), (0,)), ((), ())),
                              preferred_element_type=jnp.float32)
    o_ref[...] = (hs_ref[0] + acc)[None]


@jax.jit
def kernel(hidden_states, ln_gamma, ln_beta, router_w, router_b,
           fc1_w, fc1_b, fc2_w, fc2_b):
    hs = hidden_states.reshape(S, H)

    xn, pos, wgt, te = pl.pallas_call(
        _prologue,
        out_shape=[
            jax.ShapeDtypeStruct((S, H), jnp.bfloat16),
            jax.ShapeDtypeStruct((2, S), jnp.int32),
            jax.ShapeDtypeStruct((2, S), jnp.float32),
            jax.ShapeDtypeStruct((1, 128), jnp.int32),
        ],
    )(hs, ln_gamma.reshape(1, H), ln_beta.reshape(1, H),
      router_w, router_b.reshape(E, 1))

    grid_spec = pltpu.PrefetchScalarGridSpec(
        num_scalar_prefetch=1,
        grid=(NT,),
        in_specs=[
            pl.BlockSpec((S, H), lambda t, s: (0, 0)),
            pl.BlockSpec((2, S), lambda t, s: (0, 0)),
            pl.BlockSpec((1, I, H), lambda t, s: (s[t], 0, 0)),
            pl.BlockSpec((1, 1, I), lambda t, s: (s[t], 0, 0)),
            pl.BlockSpec((1, H, I), lambda t, s: (s[t], 0, 0)),
            pl.BlockSpec((1, 1, H), lambda t, s: (s[t], 0, 0)),
        ],
        out_specs=pl.BlockSpec((T, H), lambda t, s: (t, 0)),
        scratch_shapes=[
            pltpu.VMEM((I, H), jnp.bfloat16),
            pltpu.VMEM((H, I), jnp.bfloat16),
        ],
    )
    y = pl.pallas_call(
        _ffn,
        grid_spec=grid_spec,
        out_shape=jax.ShapeDtypeStruct((P, H), jnp.bfloat16),
    )(te.reshape(128), xn, pos, fc1_w, fc1_b.reshape(E, 1, I),
      fc2_w, fc2_b.reshape(E, 1, H))

    out = pl.pallas_call(
        _combine,
        grid=(S // TS,),
        in_specs=[
            pl.BlockSpec((1, TS, H), lambda c: (0, c, 0)),
            pl.BlockSpec((P, H), lambda c: (0, 0)),
            pl.BlockSpec((TS, 2), lambda c: (c, 0)),
            pl.BlockSpec((TS, 2), lambda c: (c, 0)),
        ],
        out_specs=pl.BlockSpec((1, TS, H), lambda c: (0, c, 0)),
        out_shape=jax.ShapeDtypeStruct((1, S, H), jnp.float32),
        compiler_params=pltpu.CompilerParams(
            dimension_semantics=("parallel",)),
    )(hidden_states, y, pos.T, wgt.T)

    return out


# speedup vs baseline: 1.0477x; 1.0477x over previous
import jax
import jax.numpy as jnp
from jax.experimental import pallas as pl
from jax.experimental.pallas import tpu as pltpu

S, H, I, E, K = 2048, 768, 3072, 8, 2
T = 128
NT = (S * K + E * (T - 1) + T - 1) // T
P = NT * T
TS = 256
NC = S // TS


def _prologue(hs_ref, g_ref, b_ref, rw_ref, rb_ref,
              xn_ref, pos_ref, wgt_ref, te_ref):
    x = hs_ref[...]
    mu = jnp.mean(x, axis=1, keepdims=True)
    xc = x - mu
    var = jnp.mean(xc * xc, axis=1, keepdims=True)
    xn = xc / jnp.sqrt(var + 1e-5) * g_ref[...] + b_ref[...]
    xn_ref[...] = xn.astype(jnp.bfloat16)

    logits = jax.lax.dot_general(
        rw_ref[...], xn, (((1,), (1,)), ((), ())),
        preferred_element_type=jnp.float32,
        precision=jax.lax.Precision.DEFAULT) + rb_ref[...]

    eio = jax.lax.broadcasted_iota(jnp.int32, (E, S), 0)
    m1 = jnp.max(logits, axis=0, keepdims=True)
    i1 = jnp.min(jnp.where(logits == m1, eio, E), axis=0, keepdims=True)
    masked = jnp.where(eio == i1, -1e30, logits)
    m2 = jnp.max(masked, axis=0, keepdims=True)
    i2 = jnp.min(jnp.where(masked == m2, eio, E), axis=0, keepdims=True)
    d = jnp.exp(m2 - m1)
    w_top1 = 1.0 / (1.0 + d)
    wgt_ref[...] = jnp.concatenate([w_top1, d * w_top1], axis=0)

    qio = jax.lax.broadcasted_iota(jnp.int32, (2 * E, S), 0)
    sel = jnp.where(qio < E,
                    jnp.broadcast_to(i1, (2 * E, S)),
                    jnp.broadcast_to(i2, (2 * E, S)))
    m = sel == qio % E

    sio = jax.lax.broadcasted_iota(jnp.int32, (S, S), 0)
    lio = jax.lax.broadcasted_iota(jnp.int32, (S, S), 1)
    tril = (sio <= lio).astype(jnp.bfloat16)
    incl = jax.lax.dot_general(m.astype(jnp.bfloat16), tril,
                               (((1,), (0,)), ((), ())),
                               preferred_element_type=jnp.float32)
    cnt_q = jnp.sum(m.astype(jnp.float32), axis=1, keepdims=True)
    cnt0 = cnt_q[0:E]
    cnts = cnt0 + cnt_q[E:]
    padded = jnp.floor((cnts + (T - 1)) / T) * T
    a8 = jax.lax.broadcasted_iota(jnp.int32, (E, E), 0)
    b8 = jax.lax.broadcasted_iota(jnp.int32, (E, E), 1)
    lstrict = (b8 < a8).astype(jnp.float32)
    start = jax.lax.dot_general(lstrict, padded, (((1,), (0,)), ((), ())),
                                preferred_element_type=jnp.float32)

    base_q = jnp.concatenate([start, start + cnt0], axis=0)
    field = jnp.where(m, base_q + incl - 1.0, 0.0)
    pos0 = jnp.sum(field[0:E], axis=0, keepdims=True)
    pos1 = jnp.sum(field[E:], axis=0, keepdims=True)
    pos_ref[...] = jnp.concatenate([pos0, pos1], axis=0).astype(jnp.int32)

    total = jnp.sum(padded, axis=0, keepdims=True) / T
    tio = (jax.lax.broadcasted_iota(jnp.int32, (1, 128), 1) * T).astype(jnp.float32)
    te = jnp.sum((start + padded <= tio).astype(jnp.int32), axis=0, keepdims=True)
    te = jnp.minimum(te, E - 1)
    lane = jax.lax.broadcasted_iota(jnp.int32, (1, 128), 1)
    te_ref[...] = jnp.where(lane == 127, total.astype(jnp.int32), te)


def _ffn_combine(s_ref, xn_ref, pos_ref, w1_ref, b1_ref, w2_ref, b2_ref,
                 hs_ref, post_ref, wgtt_ref, o_ref, y_ref):
    t = pl.program_id(0)
    used = s_ref[127]

    @pl.when(t < jnp.minimum(used, NT))
    def _ffn():
        rows = jax.lax.broadcasted_iota(jnp.int32, (T, 1), 0) + t * T
        pos0 = pos_ref[0:1, :]
        pos1 = pos_ref[1:2, :]
        oh = ((pos0 == rows) | (pos1 == rows)).astype(jnp.bfloat16)
        xg = jax.lax.dot_general(oh, xn_ref[...], (((1,), (0,)), ((), ())),
                                 preferred_element_type=jnp.float32)
        h = jax.lax.dot_general(xg.astype(jnp.bfloat16),
                                w1_ref[0].astype(jnp.bfloat16),
                                (((1,), (1,)), ((), ())),
                                preferred_element_type=jnp.float32)
        h = h + b1_ref[0]
        h = 0.5 * h * (1.0 + jax.lax.erf(h * 0.7071067811865476))
        o = jax.lax.dot_general(h.astype(jnp.bfloat16),
                                w2_ref[0].astype(jnp.bfloat16),
                                (((1,), (1,)), ((), ())),
                                preferred_element_type=jnp.float32)
        y_ref[pl.ds(t * T, T), :] = (o + b2_ref[0]).astype(jnp.bfloat16)

    @pl.when(jnp.logical_and(t >= used, t < NT))
    def _zero():
        y_ref[pl.ds(t * T, T), :] = jnp.zeros((T, H), jnp.bfloat16)

    @pl.when(t >= NT)
    def _combine():
        p0 = post_ref[:, 0:1]
        p1 = post_ref[:, 1:2]
        w0 = wgtt_ref[:, 0:1]
        w1 = wgtt_ref[:, 1:2]
        cio = jax.lax.broadcasted_iota(jnp.int32, (TS, P), 1)
        comb = jnp.where(p0 == cio, w0, 0.0) + jnp.where(p1 == cio, w1, 0.0)
        acc = jax.lax.dot_general(comb.astype(jnp.bfloat16), y_ref[...],
                                  (((1,), (0,)), ((), ())),
                                  preferred_element_type=jnp.float32)
        o_ref[...] = (hs_ref[0] + acc)[None]


@jax.jit
def kernel(hidden_states, ln_gamma, ln_beta, router_w, router_b,
           fc1_w, fc1_b, fc2_w, fc2_b):
    hs = hidden_states.reshape(S, H)

    xn, pos, wgt, te = pl.pallas_call(
        _prologue,
        out_shape=[
            jax.ShapeDtypeStruct((S, H), jnp.bfloat16),
            jax.ShapeDtypeStruct((2, S), jnp.int32),
            jax.ShapeDtypeStruct((2, S), jnp.float32),
            jax.ShapeDtypeStruct((1, 128), jnp.int32),
        ],
    )(hs, ln_gamma.reshape(1, H), ln_beta.reshape(1, H),
      router_w, router_b.reshape(E, 1))

    grid_spec = pltpu.PrefetchScalarGridSpec(
        num_scalar_prefetch=1,
        grid=(NT + NC,),
        in_specs=[
            pl.BlockSpec((S, H), lambda t, s: (0, 0)),
            pl.BlockSpec((2, S), lambda t, s: (0, 0)),
            pl.BlockSpec((1, I, H), lambda t, s: (s[jnp.minimum(t, NT - 1)], 0, 0)),
            pl.BlockSpec((1, 1, I), lambda t, s: (s[jnp.minimum(t, NT - 1)], 0, 0)),
            pl.BlockSpec((1, H, I), lambda t, s: (s[jnp.minimum(t, NT - 1)], 0, 0)),
            pl.BlockSpec((1, 1, H), lambda t, s: (s[jnp.minimum(t, NT - 1)], 0, 0)),
            pl.BlockSpec((1, TS, H), lambda t, s: (0, jnp.maximum(t - NT, 0), 0)),
            pl.BlockSpec((TS, 2), lambda t, s: (jnp.maximum(t - NT, 0), 0)),
            pl.BlockSpec((TS, 2), lambda t, s: (jnp.maximum(t - NT, 0), 0)),
        ],
        out_specs=pl.BlockSpec((1, TS, H), lambda t, s: (0, jnp.maximum(t - NT, 0), 0)),
        scratch_shapes=[pltpu.VMEM((P, H), jnp.bfloat16)],
    )
    out = pl.pallas_call(
        _ffn_combine,
        grid_spec=grid_spec,
        out_shape=jax.ShapeDtypeStruct((1, S, H), jnp.float32),
    )(te.reshape(128), xn, pos, fc1_w, fc1_b.reshape(E, 1, I),
      fc2_w, fc2_b.reshape(E, 1, H), hidden_states, pos.T, wgt.T)

    return out
